# baseline (speedup 1.0000x reference)
import functools

import jax
import jax.numpy as jnp
from jax import lax
from jax.experimental import pallas as pl
from jax.experimental.pallas import tpu as pltpu
from jax.experimental.pallas import tpu_sc as plsc

B, T, D = 8, 2048, 256
K = 1024
SPAN = 8
S = T // SPAN

_HI = jax.lax.Precision.HIGHEST


def _en_body(emb_ref, en_ref, emb2_ref):
    ones_d8 = jnp.full((D, 8), 1.0, jnp.float32)
    emb = emb_ref[...]
    en_ref[...] = jax.lax.dot_general(emb * emb, ones_d8,
                                      (((1,), (0,)), ((), ())),
                                      precision=_HI)
    e_hi = emb.astype(jnp.bfloat16).astype(jnp.float32)
    emb2_ref[:, :D] = e_hi
    emb2_ref[:, D:] = emb - e_hi


def _vq_body(x_ref, emb_ref, wb_ref, bb_ref, en8_ref, emb2_ref,
             idx_ref, bnd_ref, loss_ref, acc_ref):
    b = pl.program_id(0)

    x4 = x_ref[0]
    xb = x4.reshape(T, D)
    wb = wb_ref[...]
    bb = bb_ref[0, 0]
    emb = emb_ref[...]

    wb8 = jnp.broadcast_to(wb, (8, D))
    logits8 = jax.lax.dot_general(xb, wb8,
                                  (((1,), (1,)), ((), ()))) + bb
    bound = (logits8[:, 0:1] > 0.0).astype(jnp.float32)
    bnd_ref[0] = bound

    acc = x4[:, 0, :]
    for j in range(1, SPAN):
        acc = acc + x4[:, j, :]
    pooled = acc * (1.0 / SPAN)

    en_col = en8_ref[:, 0:1]

    dots_t = jax.lax.dot_general(emb, pooled,
                                 (((1,), (1,)), ((), ())))
    dist_t = en_col - 2.0 * dots_t

    d = dist_t
    i = jax.lax.broadcasted_iota(jnp.int32, (K, S), 0)
    w = K // 2
    while w >= 1:
        d0, d1 = d[:w, :], d[w:2 * w, :]
        i0, i1 = i[:w, :], i[w:2 * w, :]
        take = d1 < d0
        d = jnp.where(take, d1, d0)
        i = jnp.where(take, i1, i0)
        w //= 2
    idx_row = i

    k_col = jax.lax.broadcasted_iota(jnp.int32, (K, S), 0)
    onehot_t = (k_col == idx_row).astype(jnp.float32)
    qparts = jax.lax.dot_general(onehot_t, emb2_ref[...],
                                 (((0,), (0,)), ((), ())))
    quantized = qparts[:, :D] + qparts[:, D:]

    k_i = jax.lax.broadcasted_iota(jnp.int32, (K, 1), 0)
    k_hi = ((k_i // 32) * 32).astype(jnp.float32)
    k_lo = (k_i % 32).astype(jnp.float32)
    k_parts = jnp.concatenate([jnp.broadcast_to(k_hi, (K, 8)),
                               jnp.broadcast_to(k_lo, (K, 8))], axis=1)
    parts = jax.lax.dot_general(onehot_t, k_parts,
                                (((0,), (0,)), ((), ())))
    idxc = (parts[:, 0:1] + parts[:, 8:9]).astype(jnp.int32)
    idx_ref[0] = jnp.broadcast_to(idxc, (S, SPAN))

    diff2 = (quantized - pooled) ** 2
    ones_8s = jnp.full((8, S), 1.0, jnp.float32)
    row8 = jax.lax.dot_general(ones_8s, diff2,
                               (((1,), (0,)), ((), ())))
    ones_d8b = jnp.full((D, 8), 1.0, jnp.float32)
    sq11 = jax.lax.dot_general(row8, ones_d8b,
                               (((1,), (0,)), ((), ()))
                               )[0:1, 0:1]
    ones_8t = jnp.full((8, T), 1.0, jnp.float32)
    bound8 = jnp.broadcast_to(bound, (T, 8))
    sb11 = jax.lax.dot_general(ones_8t, bound8,
                               (((1,), (0,)), ((), ()))
                               )[0:1, 0:1]

    @pl.when(b == 0)
    def _():
        acc_ref[:, 0:1] = sq11
        acc_ref[:, 1:2] = sb11

    @pl.when(b > 0)
    def _():
        acc_ref[:, 0:1] += sq11
        acc_ref[:, 1:2] += sb11

    @pl.when(b == B - 1)
    def _():
        e_latent = acc_ref[:, 0:1] * (1.0 / (B * S * D))
        rate = acc_ref[:, 1:2] * (1.0 / (B * T))
        loss_ref[...] = 0.25 * e_latent + 0.01 * (rate - 1.0 / SPAN) ** 2


_NC, _NS = 2, 16
_NW = _NC * _NS
_BT = B * T
_ROWS_PER_W = _BT // _NW
_CHUNK = 128
_N_CHUNKS = _ROWS_PER_W // _CHUNK


def _sc_gather_body(table_hbm, idx_hbm, out_hbm, idx_v, rows_v, sem):
    wid = lax.axis_index("s") * _NC + lax.axis_index("c")
    base = wid * _ROWS_PER_W
    for c in range(_N_CHUNKS):
        off = base + c * _CHUNK
        pltpu.sync_copy(idx_hbm.at[pl.ds(off, _CHUNK)], idx_v)
        pltpu.async_copy(table_hbm.at[idx_v], rows_v, sem).wait()
        pltpu.sync_copy(rows_v, out_hbm.at[pl.ds(off, _CHUNK)])


_sc_gather = functools.partial(
    pl.kernel,
    out_type=jax.ShapeDtypeStruct((_BT, D), jnp.float32),
    mesh=plsc.VectorSubcoreMesh(core_axis_name="c", subcore_axis_name="s",
                                num_cores=_NC, num_subcores=_NS),
    scratch_types=[
        pltpu.VMEM((_CHUNK,), jnp.int32),
        pltpu.VMEM((_CHUNK, D), jnp.float32),
        pltpu.SemaphoreType.DMA,
    ],
)(_sc_gather_body)


@jax.jit
def kernel(x, embedding, Wb, bb):
    x4 = x.reshape(B, S, SPAN, D)
    wb2 = Wb.reshape(1, D)
    bb2 = jnp.asarray(bb, jnp.float32).reshape(1, 1)

    en8, emb2 = pl.pallas_call(
        _en_body,
        out_shape=[jax.ShapeDtypeStruct((K, 8), jnp.float32),
                   jax.ShapeDtypeStruct((K, 2 * D), jnp.float32)],
    )(embedding)

    idx3, bnd, loss = pl.pallas_call(
        _vq_body,
        grid=(B,),
        in_specs=[
            pl.BlockSpec((1, S, SPAN, D), lambda b: (b, 0, 0, 0)),
            pl.BlockSpec((K, D), lambda b: (0, 0)),
            pl.BlockSpec((1, D), lambda b: (0, 0)),
            pl.BlockSpec((1, 1), lambda b: (0, 0), memory_space=pltpu.SMEM),
            pl.BlockSpec((K, 8), lambda b: (0, 0)),
            pl.BlockSpec((K, 2 * D), lambda b: (0, 0)),
        ],
        out_specs=[
            pl.BlockSpec((1, S, SPAN), lambda b: (b, 0, 0)),
            pl.BlockSpec((1, T, 1), lambda b: (b, 0, 0)),
            pl.BlockSpec((1, 1), lambda b: (0, 0)),
        ],
        out_shape=[
            jax.ShapeDtypeStruct((B, S, SPAN), jnp.int32),
            jax.ShapeDtypeStruct((B, T, 1), jnp.float32),
            jax.ShapeDtypeStruct((1, 1), jnp.float32),
        ],
        scratch_shapes=[pltpu.VMEM((1, 2), jnp.float32)],
    )(x4, embedding, wb2, bb2, en8, emb2)

    indices_out = idx3.reshape(B, T)
    q_flat = _sc_gather(embedding, idx3.reshape(B * T))
    quantized_out = q_flat.reshape(B, T, D)
    total_loss = loss[0, 0]
    return quantized_out, total_loss, indices_out, bnd.reshape(B, T)


# speedup vs baseline: 1.6105x; 1.6105x over previous
import jax
import jax.numpy as jnp
from jax.experimental import pallas as pl
from jax.experimental.pallas import tpu as pltpu

B, T, D = 8, 2048, 256
K = 1024
SPAN = 8
S = T // SPAN
G = 2
B2, S2, T2 = B * G, S // G, T // G

_HI = jax.lax.Precision.HIGHEST


def _en_body(emb_ref, en_ref, emb2_ref):
    ones_d8 = jnp.full((D, 8), 1.0, jnp.float32)
    emb = emb_ref[...]
    en_ref[...] = jax.lax.dot_general(emb * emb, ones_d8,
                                      (((1,), (0,)), ((), ())),
                                      precision=_HI)
    e_hi = emb.astype(jnp.bfloat16).astype(jnp.float32)
    emb2_ref[:, :D] = e_hi
    emb2_ref[:, D:] = emb - e_hi


def _vq_body(x_ref, emb_ref, wb_ref, bb_ref, en8_ref, emb2_ref,
             q_ref, idx_ref, bnd_ref, loss_ref, acc_ref):
    b = pl.program_id(0)

    x4 = x_ref[0]
    xb = x4.reshape(T2, D)
    wb = wb_ref[...]
    bb = bb_ref[0, 0]
    emb = emb_ref[...]

    wb8 = jnp.broadcast_to(wb, (8, D))
    logits8 = jax.lax.dot_general(xb, wb8,
                                  (((1,), (1,)), ((), ()))) + bb
    bound = (logits8[:, 0:1] > 0.0).astype(jnp.float32)
    bnd_ref[0] = bound

    acc = x4[:, 0, :]
    for j in range(1, SPAN):
        acc = acc + x4[:, j, :]
    pooled = acc * (1.0 / SPAN)

    en_col = en8_ref[:, 0:1]

    dots_t = jax.lax.dot_general(emb, pooled,
                                 (((1,), (1,)), ((), ())))
    dist_t = en_col - 2.0 * dots_t

    d = dist_t
    i = jax.lax.broadcasted_iota(jnp.int32, (K, S2), 0)
    w = K // 2
    while w >= 1:
        d0, d1 = d[:w, :], d[w:2 * w, :]
        i0, i1 = i[:w, :], i[w:2 * w, :]
        take = d1 < d0
        d = jnp.where(take, d1, d0)
        i = jnp.where(take, i1, i0)
        w //= 2
    idx_row = i

    k_col = jax.lax.broadcasted_iota(jnp.int32, (K, S2), 0)
    onehot_t = (k_col == idx_row).astype(jnp.float32)
    qparts = jax.lax.dot_general(onehot_t, emb2_ref[...],
                                 (((0,), (0,)), ((), ())))
    quantized = qparts[:, :D] + qparts[:, D:]

    k_i = jax.lax.broadcasted_iota(jnp.int32, (K, 1), 0)
    k_hi = ((k_i // 32) * 32).astype(jnp.float32)
    k_lo = (k_i % 32).astype(jnp.float32)
    k_parts = jnp.concatenate([jnp.broadcast_to(k_hi, (K, 8)),
                               jnp.broadcast_to(k_lo, (K, 8))], axis=1)
    parts = jax.lax.dot_general(onehot_t, k_parts,
                                (((0,), (0,)), ((), ())))
    idxc = (parts[:, 0:1] + parts[:, 8:9]).astype(jnp.int32)
    idx_ref[0] = jnp.broadcast_to(idxc, (S2, SPAN))

    q_ref[0] = jnp.broadcast_to(quantized[:, None, :], (S2, SPAN, D))

    diff2 = (quantized - pooled) ** 2
    ones_8s = jnp.full((8, S2), 1.0, jnp.float32)
    row8 = jax.lax.dot_general(ones_8s, diff2,
                               (((1,), (0,)), ((), ())))
    ones_d8b = jnp.full((D, 8), 1.0, jnp.float32)
    sq11 = jax.lax.dot_general(row8, ones_d8b,
                               (((1,), (0,)), ((), ()))
                               )[0:1, 0:1]
    ones_8t = jnp.full((8, T2), 1.0, jnp.float32)
    bound8 = jnp.broadcast_to(bound, (T2, 8))
    sb11 = jax.lax.dot_general(ones_8t, bound8,
                               (((1,), (0,)), ((), ()))
                               )[0:1, 0:1]

    @pl.when(b == 0)
    def _():
        acc_ref[:, 0:1] = sq11
        acc_ref[:, 1:2] = sb11

    @pl.when(b > 0)
    def _():
        acc_ref[:, 0:1] += sq11
        acc_ref[:, 1:2] += sb11

    @pl.when(b == B2 - 1)
    def _():
        e_latent = acc_ref[:, 0:1] * (1.0 / (B * S * D))
        rate = acc_ref[:, 1:2] * (1.0 / (B * T))
        loss_ref[...] = 0.25 * e_latent + 0.01 * (rate - 1.0 / SPAN) ** 2


@jax.jit
def kernel(x, embedding, Wb, bb):
    x4 = x.reshape(B2, S2, SPAN, D)
    wb2 = Wb.reshape(1, D)
    bb2 = jnp.asarray(bb, jnp.float32).reshape(1, 1)

    en8, emb2 = pl.pallas_call(
        _en_body,
        out_shape=[jax.ShapeDtypeStruct((K, 8), jnp.float32),
                   jax.ShapeDtypeStruct((K, 2 * D), jnp.float32)],
    )(embedding)

    q4, idx3, bnd, loss = pl.pallas_call(
        _vq_body,
        grid=(B2,),
        in_specs=[
            pl.BlockSpec((1, S2, SPAN, D), lambda b: (b, 0, 0, 0)),
            pl.BlockSpec((K, D), lambda b: (0, 0)),
            pl.BlockSpec((1, D), lambda b: (0, 0)),
            pl.BlockSpec((1, 1), lambda b: (0, 0), memory_space=pltpu.SMEM),
            pl.BlockSpec((K, 8), lambda b: (0, 0)),
            pl.BlockSpec((K, 2 * D), lambda b: (0, 0)),
        ],
        out_specs=[
            pl.BlockSpec((1, S2, SPAN, D), lambda b: (b, 0, 0, 0)),
            pl.BlockSpec((1, S2, SPAN), lambda b: (b, 0, 0)),
            pl.BlockSpec((1, T2, 1), lambda b: (b, 0, 0)),
            pl.BlockSpec((1, 1), lambda b: (0, 0)),
        ],
        out_shape=[
            jax.ShapeDtypeStruct((B2, S2, SPAN, D), jnp.float32),
            jax.ShapeDtypeStruct((B2, S2, SPAN), jnp.int32),
            jax.ShapeDtypeStruct((B2, T2, 1), jnp.float32),
            jax.ShapeDtypeStruct((1, 1), jnp.float32),
        ],
        scratch_shapes=[pltpu.VMEM((1, 2), jnp.float32)],
    )(x4, embedding, wb2, bb2, en8, emb2)

    quantized_out = q4.reshape(B, T, D)
    indices_out = idx3.reshape(B, T)
    total_loss = loss[0, 0]
    return quantized_out, total_loss, indices_out, bnd.reshape(B, T)


# speedup vs baseline: 1.8857x; 1.1709x over previous
import jax
import jax.numpy as jnp
from jax.experimental import pallas as pl
from jax.experimental.pallas import tpu as pltpu

B, T, D = 8, 2048, 256
K = 1024
SPAN = 8
S = T // SPAN
G = 1
B2, S2, T2 = B * G, S // G, T // G

_HI = jax.lax.Precision.HIGHEST


def _en_body(emb_ref, en_ref, emb2_ref):
    ones_d8 = jnp.full((D, 8), 1.0, jnp.float32)
    emb = emb_ref[...]
    en_ref[...] = jax.lax.dot_general(emb * emb, ones_d8,
                                      (((1,), (0,)), ((), ())),
                                      precision=_HI)
    e_hi = emb.astype(jnp.bfloat16).astype(jnp.float32)
    emb2_ref[:, :D] = e_hi
    emb2_ref[:, D:] = emb - e_hi


def _vq_body(x_ref, emb_ref, wb_ref, bb_ref, en8_ref, emb2_ref,
             q_ref, idx_ref, bnd_ref, loss_ref, acc_ref):
    b = pl.program_id(0)

    x4 = x_ref[0]
    xb = x4.reshape(T2, D)
    wb = wb_ref[...]
    bb = bb_ref[0, 0]
    emb = emb_ref[...]

    wb8 = jnp.broadcast_to(wb, (8, D))
    logits8 = jax.lax.dot_general(xb, wb8,
                                  (((1,), (1,)), ((), ()))) + bb
    bound = (logits8[:, 0:1] > 0.0).astype(jnp.float32)
    bnd_ref[0] = bound

    acc = x4[:, 0, :]
    for j in range(1, SPAN):
        acc = acc + x4[:, j, :]
    pooled = acc * (1.0 / SPAN)

    en_col = en8_ref[:, 0:1]

    dots_t = jax.lax.dot_general(emb, pooled,
                                 (((1,), (1,)), ((), ())))
    dist_t = en_col - 2.0 * dots_t

    d = dist_t
    i = jax.lax.broadcasted_iota(jnp.int32, (K, S2), 0)
    w = K // 2
    while w >= 1:
        d0, d1 = d[:w, :], d[w:2 * w, :]
        i0, i1 = i[:w, :], i[w:2 * w, :]
        take = d1 < d0
        d = jnp.where(take, d1, d0)
        i = jnp.where(take, i1, i0)
        w //= 2
    idx_row = i

    k_col = jax.lax.broadcasted_iota(jnp.int32, (K, S2), 0)
    onehot_t = (k_col == idx_row).astype(jnp.float32)
    qparts = jax.lax.dot_general(onehot_t, emb2_ref[...],
                                 (((0,), (0,)), ((), ())))
    quantized = qparts[:, :D] + qparts[:, D:]

    k_i = jax.lax.broadcasted_iota(jnp.int32, (K, 1), 0)
    k_hi = ((k_i // 32) * 32).astype(jnp.float32)
    k_lo = (k_i % 32).astype(jnp.float32)
    k_parts = jnp.concatenate([jnp.broadcast_to(k_hi, (K, 8)),
                               jnp.broadcast_to(k_lo, (K, 8))], axis=1)
    parts = jax.lax.dot_general(onehot_t, k_parts,
                                (((0,), (0,)), ((), ())))
    idxc = (parts[:, 0:1] + parts[:, 8:9]).astype(jnp.int32)
    idx_ref[0] = jnp.broadcast_to(idxc, (S2, SPAN))

    q_ref[0] = jnp.broadcast_to(quantized[:, None, :], (S2, SPAN, D))

    diff2 = (quantized - pooled) ** 2
    ones_8s = jnp.full((8, S2), 1.0, jnp.float32)
    row8 = jax.lax.dot_general(ones_8s, diff2,
                               (((1,), (0,)), ((), ())))
    ones_d8b = jnp.full((D, 8), 1.0, jnp.float32)
    sq11 = jax.lax.dot_general(row8, ones_d8b,
                               (((1,), (0,)), ((), ()))
                               )[0:1, 0:1]
    ones_8t = jnp.full((8, T2), 1.0, jnp.float32)
    bound8 = jnp.broadcast_to(bound, (T2, 8))
    sb11 = jax.lax.dot_general(ones_8t, bound8,
                               (((1,), (0,)), ((), ()))
                               )[0:1, 0:1]

    @pl.when(b == 0)
    def _():
        acc_ref[:, 0:1] = sq11
        acc_ref[:, 1:2] = sb11

    @pl.when(b > 0)
    def _():
        acc_ref[:, 0:1] += sq11
        acc_ref[:, 1:2] += sb11

    @pl.when(b == B2 - 1)
    def _():
        e_latent = acc_ref[:, 0:1] * (1.0 / (B * S * D))
        rate = acc_ref[:, 1:2] * (1.0 / (B * T))
        loss_ref[...] = 0.25 * e_latent + 0.01 * (rate - 1.0 / SPAN) ** 2


@jax.jit
def kernel(x, embedding, Wb, bb):
    x4 = x.reshape(B2, S2, SPAN, D)
    wb2 = Wb.reshape(1, D)
    bb2 = jnp.asarray(bb, jnp.float32).reshape(1, 1)

    en8, emb2 = pl.pallas_call(
        _en_body,
        out_shape=[jax.ShapeDtypeStruct((K, 8), jnp.float32),
                   jax.ShapeDtypeStruct((K, 2 * D), jnp.float32)],
    )(embedding)

    q4, idx3, bnd, loss = pl.pallas_call(
        _vq_body,
        grid=(B2,),
        in_specs=[
            pl.BlockSpec((1, S2, SPAN, D), lambda b: (b, 0, 0, 0)),
            pl.BlockSpec((K, D), lambda b: (0, 0)),
            pl.BlockSpec((1, D), lambda b: (0, 0)),
            pl.BlockSpec((1, 1), lambda b: (0, 0), memory_space=pltpu.SMEM),
            pl.BlockSpec((K, 8), lambda b: (0, 0)),
            pl.BlockSpec((K, 2 * D), lambda b: (0, 0)),
        ],
        out_specs=[
            pl.BlockSpec((1, S2, SPAN, D), lambda b: (b, 0, 0, 0)),
            pl.BlockSpec((1, S2, SPAN), lambda b: (b, 0, 0)),
            pl.BlockSpec((1, T2, 1), lambda b: (b, 0, 0)),
            pl.BlockSpec((1, 1), lambda b: (0, 0)),
        ],
        out_shape=[
            jax.ShapeDtypeStruct((B2, S2, SPAN, D), jnp.float32),
            jax.ShapeDtypeStruct((B2, S2, SPAN), jnp.int32),
            jax.ShapeDtypeStruct((B2, T2, 1), jnp.float32),
            jax.ShapeDtypeStruct((1, 1), jnp.float32),
        ],
        scratch_shapes=[pltpu.VMEM((1, 2), jnp.float32)],
    )(x4, embedding, wb2, bb2, en8, emb2)

    quantized_out = q4.reshape(B, T, D)
    indices_out = idx3.reshape(B, T)
    total_loss = loss[0, 0]
    return quantized_out, total_loss, indices_out, bnd.reshape(B, T)
